# baseline (speedup 1.0000x reference)
import functools
import jax
import jax.numpy as jnp
from jax import lax
from jax.experimental import pallas as pl
from jax.experimental.pallas import tpu as pltpu
from jax.experimental.pallas import tpu_sc as plsc

B, L, D = 4, 2048, 1024
H, DK, DV = 16, 64, 64
BUCKET = 64
NC = L // BUCKET
NBKT = 32
EXP = 4

_INTERPRET = False


_TL1 = 256


_QKVW = 256


def _qkv_body(x_ref, wq_ref, wk_ref, wv_ref, rbig_ref, qkv_ref, bkt_ref):
    xt = x_ref[0]
    qp = jnp.dot(xt, wq_ref[...], preferred_element_type=jnp.float32)
    kp = jnp.dot(xt, wk_ref[...], preferred_element_type=jnp.float32)
    vp = jnp.dot(xt, wv_ref[...], preferred_element_type=jnp.float32)
    rot = jnp.dot(qp, rbig_ref[...], preferred_element_type=jnp.float32)
    bcols = []
    for h in range(H):
        qkv_ref[0, h, :, 0:DK] = qp[:, h * DK:(h + 1) * DK]
        qkv_ref[0, h, :, DK:2 * DK] = kp[:, h * DK:(h + 1) * DK]
        qkv_ref[0, h, :, 2 * DK:3 * DK] = vp[:, h * DV:(h + 1) * DV]
        rh = rot[:, h * 16:(h + 1) * 16]
        cat = jnp.concatenate([rh, -rh], axis=1)
        vmax = jnp.max(cat, axis=1, keepdims=True)
        lane = jax.lax.broadcasted_iota(jnp.int32, cat.shape, 1)
        first = jnp.min(jnp.where(cat == vmax, lane, NBKT), axis=1)
        bcols.append(first[:, None])
    bkt_ref[0] = jnp.concatenate(bcols, axis=1)


def _run_qkv(x, Wq, Wk, Wv, Rbig, nb=B, b_off=0):
    grid = (nb, L // _TL1)
    return pl.pallas_call(
        _qkv_body,
        grid=grid,
        in_specs=[
            pl.BlockSpec((1, _TL1, D), lambda b, t: (b + b_off, t, 0)),
            pl.BlockSpec((D, H * DK), lambda b, t: (0, 0)),
            pl.BlockSpec((D, H * DK), lambda b, t: (0, 0)),
            pl.BlockSpec((D, H * DV), lambda b, t: (0, 0)),
            pl.BlockSpec((D, H * 16), lambda b, t: (0, 0)),
        ],
        out_specs=[
            pl.BlockSpec((1, H, _TL1, _QKVW), lambda b, t: (b, 0, t, 0)),
            pl.BlockSpec((1, _TL1, H), lambda b, t: (b, t, 0)),
        ],
        out_shape=[
            jax.ShapeDtypeStruct((nb, H, L, _QKVW), jnp.float32),
            jax.ShapeDtypeStruct((nb, L, H), jnp.int32),
        ],
        interpret=_INTERPRET,
    )(x, Wq, Wk, Wv, Rbig)


_SC_NC, _SC_NS = 2, 16
_NW = _SC_NC * _SC_NS
_SEG = L // 16
_TASKS_PER_W = (B * H) // _NW


def _sc_sort_scatter(nb, off):
    mesh = plsc.VectorSubcoreMesh(core_axis_name="c", subcore_axis_name="s",
                                  num_cores=_SC_NC, num_subcores=_SC_NS)
    reps = max(1, (nb * H) // _NW)

    @functools.partial(
        pl.kernel,
        out_type=[
            jax.ShapeDtypeStruct((nb, H, L), jnp.int32),
            jax.ShapeDtypeStruct((nb, H, L, _QKVW), jnp.float32),
        ],
        mesh=mesh,
        scratch_types=[
            pltpu.VMEM((L * H,), jnp.int32),
            pltpu.VMEM((NBKT * 16,), jnp.int32),
            pltpu.VMEM((NBKT,), jnp.int32),
            pltpu.VMEM((L,), jnp.int32),
            pltpu.VMEM((L,), jnp.int32),
            pltpu.VMEM((L,), jnp.int32),
            pltpu.VMEM((_SEG, _QKVW), jnp.float32),
            pltpu.VMEM((_SEG, _QKVW), jnp.float32),
            pltpu.SemaphoreType.DMA,
            pltpu.SemaphoreType.DMA,
            pltpu.SemaphoreType.DMA,
            pltpu.SemaphoreType.DMA,
        ],
        compiler_params=pltpu.CompilerParams(needs_layout_passes=False),
        interpret=_INTERPRET,
    )
    def body(bkt_hbm, qkv_hbm,
             undo_hbm, qkvs_hbm,
             bktb_v, hist_v, offs_v, rank_v, undo_v, sidx_v,
             buf0, buf1, gsem0, gsem1, wsem0, wsem1):
        w = lax.axis_index("s") * _SC_NC + lax.axis_index("c")
        lane = lax.iota(jnp.int32, 16)
        for rep in range(reps):
            task = w * reps + rep
            bl = task // H
            b = bl + off
            h = task % H
            pltpu.sync_copy(bkt_hbm.at[b], bktb_v)
            for j in range(NBKT):
                hist_v[pl.ds(j * 16, 16)] = jnp.zeros((16,), jnp.int32)

            def pass1(t, _):
                ridx = lane * _SEG + t
                bv = plsc.load_gather(bktb_v, [ridx * H + h])
                addr = bv * 16 + lane
                cnt = plsc.load_gather(hist_v, [addr])
                plsc.store_scatter(hist_v, [addr], cnt + 1)
                plsc.store_scatter(rank_v, [ridx], cnt)
                return 0

            lax.fori_loop(0, _SEG, pass1, 0)

            run = jnp.int32(0)
            offv = [jnp.zeros((16,), jnp.int32), jnp.zeros((16,), jnp.int32)]
            for bb in range(NBKT):
                row = hist_v[pl.ds(bb * 16, 16)]
                csum = plsc.cumsum(row)
                hist_v[pl.ds(bb * 16, 16)] = csum - row
                tot = jnp.sum(row)
                offv[bb // 16] = offv[bb // 16] + jnp.where(
                    lane == (bb % 16), run, 0)
                run = run + tot
            offs_v[pl.ds(0, 16)] = offv[0]
            offs_v[pl.ds(16, 16)] = offv[1]

            def pass2(t, _):
                ridx = lane * _SEG + t
                bv = plsc.load_gather(bktb_v, [ridx * H + h])
                r = plsc.load_gather(rank_v, [ridx])
                lo = plsc.load_gather(hist_v, [bv * 16 + lane])
                bo = plsc.load_gather(offs_v, [bv])
                u = bo + lo + r
                plsc.store_scatter(undo_v, [ridx], u)
                plsc.store_scatter(sidx_v, [u], ridx)
                return 0

            lax.fori_loop(0, _SEG, pass2, 0)
            pltpu.sync_copy(undo_v, undo_hbm.at[bl, h])

            bufs = (buf0, buf1)
            gsems = (gsem0, gsem1)
            wsems = (wsem0, wsem1)

            def fire_gather(j, bi):
                return pltpu.async_copy(
                    qkv_hbm.at[b, h].at[sidx_v.at[pl.ds(j * _SEG, _SEG)]],
                    bufs[bi], gsems[bi])

            gd = {0: fire_gather(0, 0), 1: None}
            wd = {0: None, 1: None}
            for j in range(16):
                bi = j % 2
                ni = 1 - bi
                if j + 1 < 16:
                    if wd[ni] is not None:
                        wd[ni].wait()
                    gd[ni] = fire_gather(j + 1, ni)
                gd[bi].wait()
                wd[bi] = pltpu.async_copy(
                    bufs[bi], qkvs_hbm.at[bl, h, pl.ds(j * _SEG, _SEG), :],
                    wsems[bi])
            wd[0].wait()
            wd[1].wait()

    return body


def _sc_unsort_gather(nb):
    mesh = plsc.VectorSubcoreMesh(core_axis_name="c", subcore_axis_name="s",
                                  num_cores=_SC_NC, num_subcores=_SC_NS)
    reps = max(1, (nb * H) // _NW)

    @functools.partial(
        pl.kernel,
        out_type=jax.ShapeDtypeStruct((nb, H, L, _OSW), jnp.float32),
        mesh=mesh,
        scratch_types=[
            pltpu.VMEM((L,), jnp.int32),
            pltpu.VMEM((_SEG, _OSW), jnp.float32),
            pltpu.VMEM((_SEG, _OSW), jnp.float32),
            pltpu.SemaphoreType.DMA,
            pltpu.SemaphoreType.DMA,
            pltpu.SemaphoreType.DMA,
            pltpu.SemaphoreType.DMA,
        ],
        compiler_params=pltpu.CompilerParams(needs_layout_passes=False),
        interpret=_INTERPRET,
    )
    def body(os_hbm, undo_hbm, ao4_hbm, undo_v,
             buf0, buf1, gsem0, gsem1, wsem0, wsem1):
        w = lax.axis_index("s") * _SC_NC + lax.axis_index("c")
        bufs = (buf0, buf1)
        gsems = (gsem0, gsem1)
        wsems = (wsem0, wsem1)
        for rep in range(reps):
            task = w * reps + rep
            b = task // H
            h = task % H
            pltpu.sync_copy(undo_hbm.at[b, h], undo_v)

            def fire_gather(j, bi):
                return pltpu.async_copy(
                    os_hbm.at[b, h].at[undo_v.at[pl.ds(j * _SEG, _SEG)]],
                    bufs[bi], gsems[bi])

            gd = {0: fire_gather(0, 0), 1: None}
            wd = {0: None, 1: None}
            for j in range(16):
                bi = j % 2
                ni = 1 - bi
                if j + 1 < 16:
                    if wd[ni] is not None:
                        wd[ni].wait()
                    gd[ni] = fire_gather(j + 1, ni)
                gd[bi].wait()
                wd[bi] = pltpu.async_copy(
                    bufs[bi], ao4_hbm.at[b, h, pl.ds(j * _SEG, _SEG), :],
                    wsems[bi])
            wd[0].wait()
            wd[1].wait()

    return body


_OSW = 128
_TQ = 1024
_KW = _TQ + BUCKET


def _attn_body(qkvs_ref, os_ref):
    scale = 1.0 / (DK ** 0.5)
    rq = jax.lax.broadcasted_iota(jnp.int32, (_TQ, _KW), 0) // BUCKET
    rc = jax.lax.broadcasted_iota(jnp.int32, (_TQ, _KW), 1) // BUCKET
    mask = (rc == rq) | (rc == rq + 1)
    for qb in range(L // _TQ):
        base = qb * _TQ
        cur = qkvs_ref[0, 0, base:base + _TQ, :]
        pstart = (base - BUCKET) % L
        prv = qkvs_ref[0, 0, pstart:pstart + BUCKET, :]
        qc = cur[:, 0:DK]
        kwin = jnp.concatenate([prv[:, DK:2 * DK], cur[:, DK:2 * DK]], axis=0)
        vwin = jnp.concatenate([prv[:, 2 * DK:3 * DK], cur[:, 2 * DK:3 * DK]],
                               axis=0)
        dots = jax.lax.dot_general(qc, kwin, (((1,), (1,)), ((), ())),
                                   preferred_element_type=jnp.float32) * scale
        dots = jnp.where(mask, dots, -1e30)
        m = jnp.max(dots, axis=1, keepdims=True)
        e = jnp.exp(dots - m)
        s = jnp.sum(e, axis=1, keepdims=True)
        oc = jnp.dot(e / s, vwin, preferred_element_type=jnp.float32)
        os_ref[0, 0, base:base + _TQ, 0:DV] = oc


def _run_attn(qkvs):
    grid = (qkvs.shape[0], H)
    return pl.pallas_call(
        _attn_body,
        grid=grid,
        in_specs=[pl.BlockSpec((1, 1, L, _QKVW), lambda b, h: (b, h, 0, 0))],
        out_specs=pl.BlockSpec((1, 1, L, _OSW), lambda b, h: (b, h, 0, 0)),
        out_shape=jax.ShapeDtypeStruct((qkvs.shape[0], H, L, _OSW),
                                       jnp.float32),
        interpret=_INTERPRET,
    )(qkvs)


_TL2 = 256


def _tail_body(ao4_ref, x_ref, wo_ref, ln1g_ref, ln1b_ref, ln2g_ref, ln2b_ref,
               w1_ref, b1_ref, w2_ref, b2_ref, out_ref):
    ao = jnp.concatenate([ao4_ref[0, h, :, 0:DV] for h in range(H)],
                         axis=1)
    proj = jnp.dot(ao, wo_ref[...], preferred_element_type=jnp.float32)
    y = proj + x_ref[0]

    def ln(t, g, b):
        mu = jnp.mean(t, axis=1, keepdims=True)
        var = jnp.mean((t - mu) * (t - mu), axis=1, keepdims=True)
        return (t - mu) * jax.lax.rsqrt(var + 1e-5) * g + b

    x1 = ln(y, ln1g_ref[0], ln1b_ref[0])
    h1 = jnp.maximum(
        jnp.dot(x1, w1_ref[...], preferred_element_type=jnp.float32)
        + b1_ref[0], 0.0)
    y2 = jnp.dot(h1, w2_ref[...], preferred_element_type=jnp.float32) \
        + b2_ref[0] + x1
    out_ref[0] = ln(y2, ln2g_ref[0], ln2b_ref[0])


def _run_tail(ao4, x, Wo, ln1_g, ln1_b, ln2_g, ln2_b, W1, b1, W2, b2,
              b_off=0):
    nb = ao4.shape[0]
    grid = (nb, L // _TL2)
    return pl.pallas_call(
        _tail_body,
        grid=grid,
        in_specs=[
            pl.BlockSpec((1, H, _TL2, _OSW), lambda b, t: (b, 0, t, 0)),
            pl.BlockSpec((1, _TL2, D), lambda b, t: (b + b_off, t, 0)),
            pl.BlockSpec((H * DV, D), lambda b, t: (0, 0)),
            pl.BlockSpec((1, D), lambda b, t: (0, 0)),
            pl.BlockSpec((1, D), lambda b, t: (0, 0)),
            pl.BlockSpec((1, D), lambda b, t: (0, 0)),
            pl.BlockSpec((1, D), lambda b, t: (0, 0)),
            pl.BlockSpec((D, EXP * D), lambda b, t: (0, 0)),
            pl.BlockSpec((1, EXP * D), lambda b, t: (0, 0)),
            pl.BlockSpec((EXP * D, D), lambda b, t: (0, 0)),
            pl.BlockSpec((1, D), lambda b, t: (0, 0)),
        ],
        out_specs=pl.BlockSpec((1, _TL2, D), lambda b, t: (b, t, 0)),
        out_shape=jax.ShapeDtypeStruct((nb, L, D), jnp.float32),
        interpret=_INTERPRET,
    )(ao4, x, Wo, ln1_g[None], ln1_b[None], ln2_g[None], ln2_b[None],
      W1, b1[None], W2, b2[None])


def kernel(x, Wq, Wk, Wv, Wo, R, ln1_g, ln1_b, ln2_g, ln2_b, W1, b1, W2, b2):
    Rbig = jnp.kron(jnp.eye(H, dtype=jnp.float32), R)
    hb = B // 2
    qkv_a, bkt_a = _run_qkv(x, Wq, Wk, Wv, Rbig, nb=hb, b_off=0)
    qkv_b, bkt_b = _run_qkv(x, Wq, Wk, Wv, Rbig, nb=hb, b_off=hb)
    undo_a, qkvs_a = _sc_sort_scatter(hb, 0)(bkt_a.reshape(hb, L * H), qkv_a)
    undo_b, qkvs_b = _sc_sort_scatter(hb, 0)(bkt_b.reshape(hb, L * H), qkv_b)
    os_a = _run_attn(qkvs_a)
    os_b = _run_attn(qkvs_b)
    ao4_a = _sc_unsort_gather(hb)(os_a, undo_a)
    ao4_b = _sc_unsort_gather(hb)(os_b, undo_b)
    out_a = _run_tail(ao4_a, x, Wo, ln1_g, ln1_b, ln2_g, ln2_b, W1, b1, W2,
                      b2, b_off=0)
    out_b = _run_tail(ao4_b, x, Wo, ln1_g, ln1_b, ln2_g, ln2_b, W1, b1, W2,
                      b2, b_off=hb)
    return jnp.concatenate([out_a, out_b], axis=0)


# speedup vs baseline: 1.1128x; 1.1128x over previous
import functools
import jax
import jax.numpy as jnp
from jax import lax
from jax.experimental import pallas as pl
from jax.experimental.pallas import tpu as pltpu
from jax.experimental.pallas import tpu_sc as plsc

B, L, D = 4, 2048, 1024
H, DK, DV = 16, 64, 64
BUCKET = 64
NC = L // BUCKET
NBKT = 32
EXP = 4


_TL1 = 256


_QKVW = 256


def _qkv_body(x_ref, wq_ref, wk_ref, wv_ref, rbig_ref, qkv_ref, bkt_ref):
    xt = x_ref[0]
    qp = jnp.dot(xt, wq_ref[...], preferred_element_type=jnp.float32)
    kp = jnp.dot(xt, wk_ref[...], preferred_element_type=jnp.float32)
    vp = jnp.dot(xt, wv_ref[...], preferred_element_type=jnp.float32)
    rot = jnp.dot(qp, rbig_ref[...], preferred_element_type=jnp.float32)
    bcols = []
    for h in range(H):
        qkv_ref[0, h, :, 0:DK] = qp[:, h * DK:(h + 1) * DK]
        qkv_ref[0, h, :, DK:2 * DK] = kp[:, h * DK:(h + 1) * DK]
        qkv_ref[0, h, :, 2 * DK:3 * DK] = vp[:, h * DV:(h + 1) * DV]
        rh = rot[:, h * 16:(h + 1) * 16]
        cat = jnp.concatenate([rh, -rh], axis=1)
        vmax = jnp.max(cat, axis=1, keepdims=True)
        lane = jax.lax.broadcasted_iota(jnp.int32, cat.shape, 1)
        first = jnp.min(jnp.where(cat == vmax, lane, NBKT), axis=1)
        bcols.append(first[:, None])
    bkt_ref[0] = jnp.concatenate(bcols, axis=1)


def _run_qkv(x, Wq, Wk, Wv, Rbig, nb=B, b_off=0):
    grid = (nb, L // _TL1)
    return pl.pallas_call(
        _qkv_body,
        grid=grid,
        in_specs=[
            pl.BlockSpec((1, _TL1, D), lambda b, t: (b + b_off, t, 0)),
            pl.BlockSpec((D, H * DK), lambda b, t: (0, 0)),
            pl.BlockSpec((D, H * DK), lambda b, t: (0, 0)),
            pl.BlockSpec((D, H * DV), lambda b, t: (0, 0)),
            pl.BlockSpec((D, H * 16), lambda b, t: (0, 0)),
        ],
        out_specs=[
            pl.BlockSpec((1, H, _TL1, _QKVW), lambda b, t: (b, 0, t, 0)),
            pl.BlockSpec((1, _TL1, H), lambda b, t: (b, t, 0)),
        ],
        out_shape=[
            jax.ShapeDtypeStruct((nb, H, L, _QKVW), jnp.float32),
            jax.ShapeDtypeStruct((nb, L, H), jnp.int32),
        ],
    )(x, Wq, Wk, Wv, Rbig)


_SC_NC, _SC_NS = 2, 16
_NW = _SC_NC * _SC_NS
_SEG = L // 16
_TASKS_PER_W = (B * H) // _NW


def _sc_sort_scatter(nb, off):
    mesh = plsc.VectorSubcoreMesh(core_axis_name="c", subcore_axis_name="s",
                                  num_cores=_SC_NC, num_subcores=_SC_NS)
    reps = max(1, (nb * H) // _NW)

    @functools.partial(
        pl.kernel,
        out_type=[
            jax.ShapeDtypeStruct((nb, H, L), jnp.int32),
            jax.ShapeDtypeStruct((nb, H, L, _QKVW), jnp.float32),
        ],
        mesh=mesh,
        scratch_types=[
            pltpu.VMEM((L * H,), jnp.int32),
            pltpu.VMEM((NBKT * 16,), jnp.int32),
            pltpu.VMEM((NBKT,), jnp.int32),
            pltpu.VMEM((L,), jnp.int32),
            pltpu.VMEM((L,), jnp.int32),
            pltpu.VMEM((L,), jnp.int32),
            pltpu.VMEM((_SEG, _QKVW), jnp.float32),
            pltpu.VMEM((_SEG, _QKVW), jnp.float32),
            pltpu.SemaphoreType.DMA,
            pltpu.SemaphoreType.DMA,
            pltpu.SemaphoreType.DMA,
            pltpu.SemaphoreType.DMA,
        ],
        compiler_params=pltpu.CompilerParams(needs_layout_passes=False),
    )
    def body(bkt_hbm, qkv_hbm,
             undo_hbm, qkvs_hbm,
             bktb_v, hist_v, offs_v, rank_v, undo_v, sidx_v,
             buf0, buf1, gsem0, gsem1, wsem0, wsem1):
        w = lax.axis_index("s") * _SC_NC + lax.axis_index("c")
        lane = lax.iota(jnp.int32, 16)
        for rep in range(reps):
            task = w * reps + rep
            bl = task // H
            b = bl + off
            h = task % H
            pltpu.sync_copy(bkt_hbm.at[b], bktb_v)
            for j in range(NBKT):
                hist_v[pl.ds(j * 16, 16)] = jnp.zeros((16,), jnp.int32)

            def pass1(t, _):
                ridx = lane * _SEG + t
                bv = plsc.load_gather(bktb_v, [ridx * H + h])
                addr = bv * 16 + lane
                cnt = plsc.load_gather(hist_v, [addr])
                plsc.store_scatter(hist_v, [addr], cnt + 1)
                plsc.store_scatter(rank_v, [ridx], cnt)
                return 0

            lax.fori_loop(0, _SEG, pass1, 0)

            run = jnp.int32(0)
            offv = [jnp.zeros((16,), jnp.int32), jnp.zeros((16,), jnp.int32)]
            for bb in range(NBKT):
                row = hist_v[pl.ds(bb * 16, 16)]
                csum = plsc.cumsum(row)
                hist_v[pl.ds(bb * 16, 16)] = csum - row
                tot = jnp.sum(row)
                offv[bb // 16] = offv[bb // 16] + jnp.where(
                    lane == (bb % 16), run, 0)
                run = run + tot
            offs_v[pl.ds(0, 16)] = offv[0]
            offs_v[pl.ds(16, 16)] = offv[1]

            def pass2(t, _):
                ridx = lane * _SEG + t
                bv = plsc.load_gather(bktb_v, [ridx * H + h])
                r = plsc.load_gather(rank_v, [ridx])
                lo = plsc.load_gather(hist_v, [bv * 16 + lane])
                bo = plsc.load_gather(offs_v, [bv])
                u = bo + lo + r
                plsc.store_scatter(undo_v, [ridx], u)
                plsc.store_scatter(sidx_v, [u], ridx)
                return 0

            lax.fori_loop(0, _SEG, pass2, 0)
            pltpu.sync_copy(undo_v, undo_hbm.at[bl, h])

            bufs = (buf0, buf1)
            gsems = (gsem0, gsem1)
            wsems = (wsem0, wsem1)

            def fire_gather(j, bi):
                return pltpu.async_copy(
                    qkv_hbm.at[b, h].at[sidx_v.at[pl.ds(j * _SEG, _SEG)]],
                    bufs[bi], gsems[bi])

            gd = {0: fire_gather(0, 0), 1: None}
            wd = {0: None, 1: None}
            for j in range(16):
                bi = j % 2
                ni = 1 - bi
                if j + 1 < 16:
                    if wd[ni] is not None:
                        wd[ni].wait()
                    gd[ni] = fire_gather(j + 1, ni)
                gd[bi].wait()
                wd[bi] = pltpu.async_copy(
                    bufs[bi], qkvs_hbm.at[bl, h, pl.ds(j * _SEG, _SEG), :],
                    wsems[bi])
            wd[0].wait()
            wd[1].wait()

    return body


def _sc_unsort_gather(nb):
    mesh = plsc.VectorSubcoreMesh(core_axis_name="c", subcore_axis_name="s",
                                  num_cores=_SC_NC, num_subcores=_SC_NS)
    reps = max(1, (nb * H) // _NW)

    @functools.partial(
        pl.kernel,
        out_type=jax.ShapeDtypeStruct((nb, H, L, _OSW), jnp.float32),
        mesh=mesh,
        scratch_types=[
            pltpu.VMEM((L,), jnp.int32),
            pltpu.VMEM((_SEG, _OSW), jnp.float32),
            pltpu.VMEM((_SEG, _OSW), jnp.float32),
            pltpu.SemaphoreType.DMA,
            pltpu.SemaphoreType.DMA,
            pltpu.SemaphoreType.DMA,
            pltpu.SemaphoreType.DMA,
        ],
        compiler_params=pltpu.CompilerParams(needs_layout_passes=False),
    )
    def body(os_hbm, undo_hbm, ao4_hbm, undo_v,
             buf0, buf1, gsem0, gsem1, wsem0, wsem1):
        w = lax.axis_index("s") * _SC_NC + lax.axis_index("c")
        bufs = (buf0, buf1)
        gsems = (gsem0, gsem1)
        wsems = (wsem0, wsem1)
        for rep in range(reps):
            task = w * reps + rep
            b = task // H
            h = task % H
            pltpu.sync_copy(undo_hbm.at[b, h], undo_v)

            def fire_gather(j, bi):
                return pltpu.async_copy(
                    os_hbm.at[b, h].at[undo_v.at[pl.ds(j * _SEG, _SEG)]],
                    bufs[bi], gsems[bi])

            gd = {0: fire_gather(0, 0), 1: None}
            wd = {0: None, 1: None}
            for j in range(16):
                bi = j % 2
                ni = 1 - bi
                if j + 1 < 16:
                    if wd[ni] is not None:
                        wd[ni].wait()
                    gd[ni] = fire_gather(j + 1, ni)
                gd[bi].wait()
                wd[bi] = pltpu.async_copy(
                    bufs[bi], ao4_hbm.at[b, h, pl.ds(j * _SEG, _SEG), :],
                    wsems[bi])
            wd[0].wait()
            wd[1].wait()

    return body


_OSW = 128
_TQ = 512
_KW = _TQ + BUCKET


def _attn_body(qkvs_ref, os_ref):
    scale = 1.0 / (DK ** 0.5)
    rq = jax.lax.broadcasted_iota(jnp.int32, (_TQ, _KW), 0) // BUCKET
    rc = jax.lax.broadcasted_iota(jnp.int32, (_TQ, _KW), 1) // BUCKET
    mask = (rc == rq) | (rc == rq + 1)
    for qb in range(L // _TQ):
        base = qb * _TQ
        cur = qkvs_ref[0, 0, base:base + _TQ, :]
        pstart = (base - BUCKET) % L
        prv = qkvs_ref[0, 0, pstart:pstart + BUCKET, :]
        qc = cur[:, 0:DK]
        kwin = jnp.concatenate([prv[:, DK:2 * DK], cur[:, DK:2 * DK]], axis=0)
        vwin = jnp.concatenate([prv[:, 2 * DK:3 * DK], cur[:, 2 * DK:3 * DK]],
                               axis=0)
        dots = jax.lax.dot_general(qc, kwin, (((1,), (1,)), ((), ())),
                                   preferred_element_type=jnp.float32) * scale
        dots = jnp.where(mask, dots, -1e30)
        m = jnp.max(dots, axis=1, keepdims=True)
        e = jnp.exp(dots - m)
        s = jnp.sum(e, axis=1, keepdims=True)
        oc = jnp.dot(e / s, vwin, preferred_element_type=jnp.float32)
        os_ref[0, 0, base:base + _TQ, 0:DV] = oc


def _run_attn(qkvs):
    grid = (qkvs.shape[0], H)
    return pl.pallas_call(
        _attn_body,
        grid=grid,
        in_specs=[pl.BlockSpec((1, 1, L, _QKVW), lambda b, h: (b, h, 0, 0))],
        out_specs=pl.BlockSpec((1, 1, L, _OSW), lambda b, h: (b, h, 0, 0)),
        out_shape=jax.ShapeDtypeStruct((qkvs.shape[0], H, L, _OSW),
                                       jnp.float32),
    )(qkvs)


_TL2 = 256


def _tail_body(ao4_ref, x_ref, wo_ref, ln1g_ref, ln1b_ref, ln2g_ref, ln2b_ref,
               w1_ref, b1_ref, w2_ref, b2_ref, out_ref):
    ao = jnp.concatenate([ao4_ref[0, h, :, 0:DV] for h in range(H)],
                         axis=1)
    proj = jnp.dot(ao, wo_ref[...], preferred_element_type=jnp.float32)
    y = proj + x_ref[0]

    def ln(t, g, b):
        mu = jnp.mean(t, axis=1, keepdims=True)
        var = jnp.mean((t - mu) * (t - mu), axis=1, keepdims=True)
        return (t - mu) * jax.lax.rsqrt(var + 1e-5) * g + b

    x1 = ln(y, ln1g_ref[0], ln1b_ref[0])
    h1 = jnp.maximum(
        jnp.dot(x1, w1_ref[...], preferred_element_type=jnp.float32)
        + b1_ref[0], 0.0)
    y2 = jnp.dot(h1, w2_ref[...], preferred_element_type=jnp.float32) \
        + b2_ref[0] + x1
    out_ref[0] = ln(y2, ln2g_ref[0], ln2b_ref[0])


def _run_tail(ao4, x, Wo, ln1_g, ln1_b, ln2_g, ln2_b, W1, b1, W2, b2,
              b_off=0):
    nb = ao4.shape[0]
    grid = (nb, L // _TL2)
    return pl.pallas_call(
        _tail_body,
        grid=grid,
        in_specs=[
            pl.BlockSpec((1, H, _TL2, _OSW), lambda b, t: (b, 0, t, 0)),
            pl.BlockSpec((1, _TL2, D), lambda b, t: (b + b_off, t, 0)),
            pl.BlockSpec((H * DV, D), lambda b, t: (0, 0)),
            pl.BlockSpec((1, D), lambda b, t: (0, 0)),
            pl.BlockSpec((1, D), lambda b, t: (0, 0)),
            pl.BlockSpec((1, D), lambda b, t: (0, 0)),
            pl.BlockSpec((1, D), lambda b, t: (0, 0)),
            pl.BlockSpec((D, EXP * D), lambda b, t: (0, 0)),
            pl.BlockSpec((1, EXP * D), lambda b, t: (0, 0)),
            pl.BlockSpec((EXP * D, D), lambda b, t: (0, 0)),
            pl.BlockSpec((1, D), lambda b, t: (0, 0)),
        ],
        out_specs=pl.BlockSpec((1, _TL2, D), lambda b, t: (b, t, 0)),
        out_shape=jax.ShapeDtypeStruct((nb, L, D), jnp.float32),
    )(ao4, x, Wo, ln1_g[None], ln1_b[None], ln2_g[None], ln2_b[None],
      W1, b1[None], W2, b2[None])


def kernel(x, Wq, Wk, Wv, Wo, R, ln1_g, ln1_b, ln2_g, ln2_b, W1, b1, W2, b2):
    Rbig = jnp.kron(jnp.eye(H, dtype=jnp.float32), R)
    hb = B // 2
    qkv_a, bkt_a = _run_qkv(x, Wq, Wk, Wv, Rbig, nb=hb, b_off=0)
    qkv_b, bkt_b = _run_qkv(x, Wq, Wk, Wv, Rbig, nb=hb, b_off=hb)
    undo_a, qkvs_a = _sc_sort_scatter(hb, 0)(bkt_a.reshape(hb, L * H), qkv_a)
    undo_b, qkvs_b = _sc_sort_scatter(hb, 0)(bkt_b.reshape(hb, L * H), qkv_b)
    os_a = _run_attn(qkvs_a)
    os_b = _run_attn(qkvs_b)
    ao4_a = _sc_unsort_gather(hb)(os_a, undo_a)
    ao4_b = _sc_unsort_gather(hb)(os_b, undo_b)
    out_a = _run_tail(ao4_a, x, Wo, ln1_g, ln1_b, ln2_g, ln2_b, W1, b1, W2,
                      b2, b_off=0)
    out_b = _run_tail(ao4_b, x, Wo, ln1_g, ln1_b, ln2_g, ln2_b, W1, b1, W2,
                      b2, b_off=hb)
    return jnp.concatenate([out_a, out_b], axis=0)
